# baseline (speedup 1.0000x reference)
import jax
import jax.numpy as jnp
from jax.experimental import pallas as pl

_NUM_PHN = 39
_KP = 128


def _body(f_ref, sc_ref, phc_ref, out_ref):
    F = f_ref[...]
    sc = sc_ref[...]
    phc = phc_ref[...]
    N, D = F.shape

    phc_s = jnp.minimum(jnp.maximum(phc, 0), _NUM_PHN - 1)
    lane = jax.lax.broadcasted_iota(jnp.int32, (N, _KP), 1)
    E = (lane == phc_s).astype(jnp.float32)

    valid = sc >= 0.0
    validf = jnp.where(valid, 1.0, 0.0)
    m_high = jnp.where(valid & (sc == 2.0), 1.0, 0.0)
    high_hits = jax.lax.dot_general(
        E, m_high, (((0,), (0,)), ((), ())),
        preferred_element_type=jnp.float32,
    )
    hp_row = jnp.transpose(jnp.where(high_hits > 0.0, 1.0, 0.0))
    keepf = validf * jnp.sum(E * hp_row, axis=1, keepdims=True)
    Ek = E * keepf
    counts = jax.lax.dot_general(
        Ek, jnp.ones((N, 1), jnp.float32), (((0,), (0,)), ((), ())),
        preferred_element_type=jnp.float32,
    )
    presentf = jnp.where(counts > 0.0, 1.0, 0.0)
    n_u = jnp.sum(presentf)

    center = jax.lax.dot_general(
        Ek, F, (((0,), (0,)), ((), ())), preferred_element_type=jnp.float32
    ) / jnp.maximum(counts, 1.0)
    cn = jnp.sqrt(jnp.sum(center * center, axis=1, keepdims=True))
    center = center / jnp.maximum(cn, 1e-12)
    pn = jnp.sqrt(jnp.sum(center * center, axis=1, keepdims=True))
    p = center / jnp.maximum(pn, 1e-12)

    pn2 = jnp.sum(p * p, axis=1, keepdims=True)
    Gpp = jax.lax.dot_general(
        p, p, (((1,), (1,)), ((), ())), preferred_element_type=jnp.float32
    )
    ii = jax.lax.broadcasted_iota(jnp.int32, (_KP, _KP), 0)
    jj = jax.lax.broadcasted_iota(jnp.int32, (_KP, _KP), 1)
    eye = jnp.where(ii == jj, 1.0, 0.0)
    pn2_row = jnp.sum(Gpp * eye, axis=0, keepdims=True)
    d2 = pn2 + pn2_row - 2.0 * Gpp
    dist = jnp.sqrt(jnp.maximum(d2, 1e-12))
    pair_present = jax.lax.dot_general(
        presentf, presentf, (((1,), (1,)), ((), ())),
        preferred_element_type=jnp.float32,
    )
    pair_mask = (pair_present > 0.5) & (ii < jj)
    denom = jnp.maximum(n_u * (n_u - 1.0) * 0.5, 1.0)
    diversity = jnp.sum(jnp.where(pair_mask, dist, 0.0)) / denom

    fn2 = jnp.sum(F * F, axis=1, keepdims=True)
    rs = 1.0 / jnp.maximum(jnp.sqrt(fn2), 1e-12)
    fhat = F * rs
    hn2 = fn2 * (rs * rs)
    G = jax.lax.dot_general(
        fhat, p, (((1,), (1,)), ((), ())), preferred_element_type=jnp.float32
    )
    pn2_l = jnp.transpose(pn2)
    dsq = hn2 + jnp.sum(Ek * (pn2_l - 2.0 * G), axis=1, keepdims=True)
    nzf = keepf * jnp.where(dsq > 0.0, 1.0, 0.0)
    cnt = jnp.sum(nzf)
    w = 3.0 - sc
    tsum = jnp.sum(nzf * jnp.sqrt(jnp.maximum(dsq, 0.0)) * w)
    tightness = tsum / jnp.maximum(cnt, 1.0)

    loss = 0.1 * tightness - 0.5 * diversity
    out_ref[...] = jnp.broadcast_to(jnp.where(n_u >= 2.0, loss, 0.0), (1, 1))


def kernel(features, scores, phn_ids):
    B, T, D = features.shape
    N = B * T
    F = features.reshape(N, D)
    sc = scores.reshape(N, 1)
    phc = phn_ids.reshape(N, 1).astype(jnp.int32)
    out = pl.pallas_call(
        _body,
        out_shape=jax.ShapeDtypeStruct((1, 1), jnp.float32),
    )(F, sc, phc)
    return out[0, 0]
